# baseline (speedup 1.0000x reference)
import functools

import jax
import jax.numpy as jnp
from jax import lax
from jax.experimental import pallas as pl
from jax.experimental.pallas import tpu as pltpu
from jax.experimental.pallas import tpu_sc as plsc

NSAMPLE = 16
C = 256
B, N1, N2 = 4, 1024, 4096
Q = 512
NB = N2 // Q

NC, NS = 2, 16
NW = NC * NS
ROWS = NSAMPLE * N2
RPW = ROWS // NW
CHUNK = 128
NCHUNK = RPW // CHUNK


def _dot(a, b):
    return jax.lax.dot_general(a, b, (((1,), (0,)), ((), ())),
                               preferred_element_type=jnp.float32)


def _prep_body(feat1_ref, feat2_ref, value1_ref, x1t_ref, x2t_ref, w1a_ref,
               w1b_ref, w1ct_ref, b1_ref, tab_ref, a2t_ref):
    f1 = feat1_ref[0]
    f2 = feat2_ref[0]
    w1ct = w1ct_ref[...]
    a1 = jax.lax.dot_general(
        f1, w1a_ref[...], (((0,), (1,)), ((), ())),
        preferred_element_type=jnp.float32) + _dot(x1t_ref[0], w1ct)
    vt = jnp.transpose(value1_ref[0], (1, 0))
    a1b = lax.bitcast_convert_type(
        a1.astype(jnp.bfloat16).astype(jnp.float32), jnp.uint32)
    vtb = lax.bitcast_convert_type(
        vt.astype(jnp.bfloat16).astype(jnp.float32), jnp.uint32)
    packed = (a1b >> 16) | (vtb & jnp.uint32(0xFFFF0000))
    tab_ref[0] = lax.bitcast_convert_type(packed, jnp.int32)
    a2t_ref[0] = (jax.lax.dot_general(
        f2, w1b_ref[...], (((0,), (1,)), ((), ())),
        preferred_element_type=jnp.float32)
        + b1_ref[...] - _dot(x2t_ref[0], w1ct))


def _knn_body(xyz1_ref, xyz2t_ref, idx_ref):
    q = xyz2t_ref[0]
    t = xyz1_ref[0]
    d = jnp.zeros((Q, N1), jnp.float32)
    for dim in range(3):
        diff = q[:, dim:dim + 1] - t[dim:dim + 1, :]
        d = d + diff * diff

    iota = lax.broadcasted_iota(jnp.int32, (Q, N1), 1).astype(jnp.float32)
    kiota = lax.broadcasted_iota(jnp.int32, (Q, NSAMPLE), 1)
    acc = jnp.zeros((Q, NSAMPLE), jnp.float32)
    for _k in range(NSAMPLE):
        m = jnp.min(d, axis=1, keepdims=True)
        eq = d == m
        idxq = jnp.sum(jnp.where(eq, iota, 0.0), axis=1,
                       keepdims=True)
        d = jnp.where(eq, jnp.inf, d)
        acc = jnp.where(kiota == _k, idxq, acc)
    idx_ref[0] = jnp.transpose(acc, (1, 0)).astype(jnp.int32)


def _sc_gather(tab_hbm, idx_hbm, out_hbm, idx_v, rows_v0, rows_v1,
               sem_g0, sem_g1, sem_w0, sem_w1):
    wid = lax.axis_index("s") * NC + lax.axis_index("c")
    base = wid * RPW
    rows_v = (rows_v0, rows_v1)
    sem_g = (sem_g0, sem_g1)
    sem_w = (sem_w0, sem_w1)
    pltpu.sync_copy(idx_hbm.at[pl.ds(base, RPW)], idx_v)
    pending = [None, None]
    for i in range(NCHUNK):
        j = i % 2
        if pending[j] is not None:
            pending[j].wait()
        off = base + i * CHUNK
        pltpu.async_copy(tab_hbm.at[idx_v.at[pl.ds(i * CHUNK, CHUNK)]],
                         rows_v[j], sem_g[j]).wait()
        pending[j] = pltpu.async_copy(rows_v[j], out_hbm.at[pl.ds(off, CHUNK)],
                                      sem_w[j])
    for h in pending:
        if h is not None:
            h.wait()


def _mlp_body(g_ref, a2t_ref, w2t_ref, b2_ref, out_ref):
    a2 = a2t_ref[0]
    w2t = w2t_ref[...]
    b2 = b2_ref[...]
    num = jnp.zeros((Q, C), jnp.float32)
    den = jnp.zeros((Q, C), jnp.float32)
    hi_mask = jnp.int32(-65536)
    for k in range(NSAMPLE):
        gk = g_ref[0, k]
        ga = lax.bitcast_convert_type(lax.shift_left(gk, 16), jnp.float32)
        gv = lax.bitcast_convert_type(gk & hi_mask, jnp.float32)
        h = ga + a2
        h = jnp.maximum(h, 0.1 * h)
        e = jnp.exp(_dot(h.astype(jnp.bfloat16), w2t) + b2)
        num = num + e * gv
        den = den + e
    out_ref[0] = jnp.transpose(num / den, (1, 0))


@jax.jit
def _run(xyz1, xyz2, feat1, feat2, value1, W1, b1, W2, b2):
    w1a = W1[:, :C]
    w1b = W1[:, C:2 * C]
    w1ct = jnp.zeros((8, C), jnp.float32).at[:3, :].set(W1[:, 2 * C:].T)
    xyz1t = jnp.concatenate(
        [jnp.transpose(xyz1, (0, 2, 1)),
         jnp.zeros((B, N1, 5), jnp.float32)], axis=-1)
    xyz2t = jnp.concatenate(
        [jnp.transpose(xyz2, (0, 2, 1)),
         jnp.zeros((B, N2, 5), jnp.float32)], axis=-1)

    tab, a2t = pl.pallas_call(
        _prep_body,
        grid=(B,),
        in_specs=[
            pl.BlockSpec((1, C, N1), lambda b: (b, 0, 0)),
            pl.BlockSpec((1, C, N2), lambda b: (b, 0, 0)),
            pl.BlockSpec((1, C, N1), lambda b: (b, 0, 0)),
            pl.BlockSpec((1, N1, 8), lambda b: (b, 0, 0)),
            pl.BlockSpec((1, N2, 8), lambda b: (b, 0, 0)),
            pl.BlockSpec((C, C), lambda b: (0, 0)),
            pl.BlockSpec((C, C), lambda b: (0, 0)),
            pl.BlockSpec((8, C), lambda b: (0, 0)),
            pl.BlockSpec((1, C), lambda b: (0, 0)),
        ],
        out_specs=[
            pl.BlockSpec((1, N1, C), lambda b: (b, 0, 0)),
            pl.BlockSpec((1, N2, C), lambda b: (b, 0, 0)),
        ],
        out_shape=[
            jax.ShapeDtypeStruct((B, N1, C), jnp.int32),
            jax.ShapeDtypeStruct((B, N2, C), jnp.float32),
        ],
    )(feat1, feat2, value1, xyz1t, xyz2t, w1a, w1b, w1ct, b1[None, :])

    gather = pl.kernel(
        _sc_gather,
        out_type=jax.ShapeDtypeStruct((ROWS, C), jnp.int32),
        mesh=plsc.VectorSubcoreMesh(core_axis_name="c", subcore_axis_name="s"),
        scratch_types=[
            pltpu.VMEM((RPW,), jnp.int32),
            pltpu.VMEM((CHUNK, C), jnp.int32),
            pltpu.VMEM((CHUNK, C), jnp.int32),
            pltpu.SemaphoreType.DMA,
            pltpu.SemaphoreType.DMA,
            pltpu.SemaphoreType.DMA,
            pltpu.SemaphoreType.DMA,
        ],
    )

    idx = pl.pallas_call(
        _knn_body,
        grid=(B, NB),
        in_specs=[
            pl.BlockSpec((1, 3, N1), lambda bb, i: (bb, 0, 0)),
            pl.BlockSpec((1, Q, 8), lambda bb, i: (bb, i, 0)),
        ],
        out_specs=pl.BlockSpec((1, NSAMPLE, Q), lambda bb, i: (bb, 0, i)),
        out_shape=jax.ShapeDtypeStruct((B, NSAMPLE, N2), jnp.int32),
    )(xyz1, xyz2t)

    outs = []
    for b in range(B):
        g_b = gather(tab[b], idx[b].reshape(ROWS))

        out_b = pl.pallas_call(
            _mlp_body,
            grid=(1, NB),
            in_specs=[
                pl.BlockSpec((1, NSAMPLE, Q, C), lambda bb, i: (bb, 0, i, 0)),
                pl.BlockSpec((1, Q, C), lambda bb, i: (bb, i, 0)),
                pl.BlockSpec((C, C), lambda bb, i: (0, 0)),
                pl.BlockSpec((1, C), lambda bb, i: (0, 0)),
            ],
            out_specs=pl.BlockSpec((1, C, Q), lambda bb, i: (bb, 0, i)),
            out_shape=jax.ShapeDtypeStruct((1, C, N2), jnp.float32),
        )(g_b.reshape(1, NSAMPLE, N2, C), a2t[b:b + 1],
          W2.T.astype(jnp.bfloat16), b2[None, :])
        outs.append(out_b)

    outt = jnp.concatenate(outs, axis=0)
    return outt


def kernel(xyz1, xyz2, feat1, feat2, value1, W1, b1, W2, b2):
    return _run(xyz1, xyz2, feat1, feat2, value1, W1, b1, W2, b2)


# speedup vs baseline: 1.2489x; 1.2489x over previous
import functools

import jax
import jax.numpy as jnp
from jax import lax
from jax.experimental import pallas as pl
from jax.experimental.pallas import tpu as pltpu
from jax.experimental.pallas import tpu_sc as plsc

NSAMPLE = 16
C = 256
B, N1, N2 = 4, 1024, 4096
Q = 512
NB = N2 // Q

NC, NS = 2, 16
NW = NC * NS
ROWS = NSAMPLE * N2
RPW = ROWS // NW
CHUNK = 256
NCHUNK = RPW // CHUNK


def _dot(a, b):
    return jax.lax.dot_general(a, b, (((1,), (0,)), ((), ())),
                               preferred_element_type=jnp.float32)


def _prep_body(feat1_ref, feat2_ref, value1_ref, x1t_ref, x2t_ref, w1a_ref,
               w1b_ref, w1ct_ref, b1_ref, tab_ref, a2t_ref):
    f1 = feat1_ref[0]
    f2 = feat2_ref[0]
    w1ct = w1ct_ref[...]
    a1 = jax.lax.dot_general(
        f1, w1a_ref[...], (((0,), (1,)), ((), ())),
        preferred_element_type=jnp.float32) + _dot(x1t_ref[0], w1ct)
    vt = jnp.transpose(value1_ref[0], (1, 0))
    a1b = lax.bitcast_convert_type(
        a1.astype(jnp.bfloat16).astype(jnp.float32), jnp.uint32)
    vtb = lax.bitcast_convert_type(
        vt.astype(jnp.bfloat16).astype(jnp.float32), jnp.uint32)
    packed = (a1b >> 16) | (vtb & jnp.uint32(0xFFFF0000))
    tab_ref[0] = lax.bitcast_convert_type(packed, jnp.int32)
    a2t_ref[0] = (jax.lax.dot_general(
        f2, w1b_ref[...], (((0,), (1,)), ((), ())),
        preferred_element_type=jnp.float32)
        + b1_ref[...] - _dot(x2t_ref[0], w1ct))


def _knn_body(xyz1_ref, xyz2t_ref, idx_ref):
    q = xyz2t_ref[0]
    t = xyz1_ref[0]
    d = jnp.zeros((Q, N1), jnp.float32)
    for dim in range(3):
        diff = q[:, dim:dim + 1] - t[dim:dim + 1, :]
        d = d + diff * diff

    iota = lax.broadcasted_iota(jnp.int32, (Q, N1), 1).astype(jnp.float32)
    kiota = lax.broadcasted_iota(jnp.int32, (Q, NSAMPLE), 1)
    acc = jnp.zeros((Q, NSAMPLE), jnp.float32)
    for _k in range(NSAMPLE):
        m = jnp.min(d, axis=1, keepdims=True)
        eq = d == m
        idxq = jnp.sum(jnp.where(eq, iota, 0.0), axis=1,
                       keepdims=True)
        d = jnp.where(eq, jnp.inf, d)
        acc = jnp.where(kiota == _k, idxq, acc)
    idx_ref[0] = jnp.transpose(acc, (1, 0)).astype(jnp.int32)


def _sc_gather(tab_hbm, idx_hbm, out_hbm, idx_v, rows_v, sem):
    wid = lax.axis_index("s") * NC + lax.axis_index("c")
    base = wid * RPW

    def body(i, carry):
        off = base + i * CHUNK
        pltpu.sync_copy(idx_hbm.at[pl.ds(off, CHUNK)], idx_v)
        pltpu.async_copy(tab_hbm.at[idx_v], rows_v, sem).wait()
        pltpu.sync_copy(rows_v, out_hbm.at[pl.ds(off, CHUNK)])
        return carry

    lax.fori_loop(0, NCHUNK, body, 0)


def _mlp_body(g_ref, a2t_ref, w2t_ref, b2_ref, out_ref):
    a2 = a2t_ref[0]
    w2t = w2t_ref[...]
    b2 = b2_ref[...]
    num = jnp.zeros((Q, C), jnp.float32)
    den = jnp.zeros((Q, C), jnp.float32)
    hi_mask = jnp.int32(-65536)
    for k in range(NSAMPLE):
        gk = g_ref[0, k]
        ga = lax.bitcast_convert_type(lax.shift_left(gk, 16), jnp.float32)
        gv = lax.bitcast_convert_type(gk & hi_mask, jnp.float32)
        h = ga + a2
        h = jnp.maximum(h, 0.1 * h)
        e = jnp.exp(_dot(h.astype(jnp.bfloat16), w2t) + b2)
        num = num + e * gv
        den = den + e
    out_ref[0] = jnp.transpose(num / den, (1, 0))


@jax.jit
def _run(xyz1, xyz2, feat1, feat2, value1, W1, b1, W2, b2):
    w1a = W1[:, :C]
    w1b = W1[:, C:2 * C]
    w1ct = jnp.zeros((8, C), jnp.float32).at[:3, :].set(W1[:, 2 * C:].T)
    xyz1t = jnp.concatenate(
        [jnp.transpose(xyz1, (0, 2, 1)),
         jnp.zeros((B, N1, 5), jnp.float32)], axis=-1)
    xyz2t = jnp.concatenate(
        [jnp.transpose(xyz2, (0, 2, 1)),
         jnp.zeros((B, N2, 5), jnp.float32)], axis=-1)

    tab, a2t = pl.pallas_call(
        _prep_body,
        grid=(B,),
        in_specs=[
            pl.BlockSpec((1, C, N1), lambda b: (b, 0, 0)),
            pl.BlockSpec((1, C, N2), lambda b: (b, 0, 0)),
            pl.BlockSpec((1, C, N1), lambda b: (b, 0, 0)),
            pl.BlockSpec((1, N1, 8), lambda b: (b, 0, 0)),
            pl.BlockSpec((1, N2, 8), lambda b: (b, 0, 0)),
            pl.BlockSpec((C, C), lambda b: (0, 0)),
            pl.BlockSpec((C, C), lambda b: (0, 0)),
            pl.BlockSpec((8, C), lambda b: (0, 0)),
            pl.BlockSpec((1, C), lambda b: (0, 0)),
        ],
        out_specs=[
            pl.BlockSpec((1, N1, C), lambda b: (b, 0, 0)),
            pl.BlockSpec((1, N2, C), lambda b: (b, 0, 0)),
        ],
        out_shape=[
            jax.ShapeDtypeStruct((B, N1, C), jnp.int32),
            jax.ShapeDtypeStruct((B, N2, C), jnp.float32),
        ],
    )(feat1, feat2, value1, xyz1t, xyz2t, w1a, w1b, w1ct, b1[None, :])

    gather = pl.kernel(
        _sc_gather,
        out_type=jax.ShapeDtypeStruct((ROWS, C), jnp.int32),
        mesh=plsc.VectorSubcoreMesh(core_axis_name="c", subcore_axis_name="s"),
        scratch_types=[
            pltpu.VMEM((CHUNK,), jnp.int32),
            pltpu.VMEM((CHUNK, C), jnp.int32),
            pltpu.SemaphoreType.DMA,
        ],
    )

    outs = []
    for b in range(B):
        idx_b = pl.pallas_call(
            _knn_body,
            grid=(1, NB),
            in_specs=[
                pl.BlockSpec((1, 3, N1), lambda bb, i: (bb, 0, 0)),
                pl.BlockSpec((1, Q, 8), lambda bb, i: (bb, i, 0)),
            ],
            out_specs=pl.BlockSpec((1, NSAMPLE, Q), lambda bb, i: (bb, 0, i)),
            out_shape=jax.ShapeDtypeStruct((1, NSAMPLE, N2), jnp.int32),
        )(xyz1[b:b + 1], xyz2t[b:b + 1])

        g_b = gather(tab[b], idx_b.reshape(ROWS))

        out_b = pl.pallas_call(
            _mlp_body,
            grid=(1, NB),
            in_specs=[
                pl.BlockSpec((1, NSAMPLE, Q, C), lambda bb, i: (bb, 0, i, 0)),
                pl.BlockSpec((1, Q, C), lambda bb, i: (bb, i, 0)),
                pl.BlockSpec((C, C), lambda bb, i: (0, 0)),
                pl.BlockSpec((1, C), lambda bb, i: (0, 0)),
            ],
            out_specs=pl.BlockSpec((1, C, Q), lambda bb, i: (bb, 0, i)),
            out_shape=jax.ShapeDtypeStruct((1, C, N2), jnp.float32),
        )(g_b.reshape(1, NSAMPLE, N2, C), a2t[b:b + 1],
          W2.T.astype(jnp.bfloat16), b2[None, :])
        outs.append(out_b)

    outt = jnp.concatenate(outs, axis=0)
    return outt


def kernel(xyz1, xyz2, feat1, feat2, value1, W1, b1, W2, b2):
    return _run(xyz1, xyz2, feat1, feat2, value1, W1, b1, W2, b2)
